# Initial kernel scaffold; baseline (speedup 1.0000x reference)
import jax
import jax.numpy as jnp
from jax.experimental import pallas as pl


def kernel(inputs, edge_index, W_self1, W_neigh1, b1, W_self2, W_neigh2, b2):
    raise NotImplementedError("write your pallas kernel here")


# speedup vs baseline: 6.3259x; 6.3259x over previous
import jax
import jax.numpy as jnp
from jax import lax
from jax.experimental import pallas as pl
from jax.experimental.pallas import tpu as pltpu
from jax.experimental.pallas import tpu_sc as plsc

NC = 2
NS = 16
NW = NC * NS
C = 128
PAD_SPREAD = 64

_HIGH = jax.lax.Precision.HIGHEST


def _dot(a, b):
    return jax.lax.dot_general(a, b, (((1,), (0,)), ((), ())),
                               precision=_HIGH,
                               preferred_element_type=jnp.float32)


def _make_sc_accum(n_nodes, d, nch):
    na = n_nodes + PAD_SPREAD
    assert na % NS == 0 and d % 16 == 0 and (d * 4) % 64 == 0
    rps_acc = na // NS
    rps_out = (n_nodes // NS) & ~7
    rem_out = n_nodes - NS * rps_out
    mesh = plsc.VectorSubcoreMesh(core_axis_name="c", subcore_axis_name="s",
                                  num_cores=NC, num_subcores=NS)

    def body(table, srcs, dsts, acc_out, src_v, dst_v, rows_v, sem, acc_sh):
        c = lax.axis_index("c")
        s = lax.axis_index("s")
        wid = s * NC + c
        z16 = jnp.zeros((16,), jnp.float32)

        @pl.loop(0, C)
        def _fill(r):
            for k in range(d // 16):
                rows_v[r, pl.ds(k * 16, 16)] = z16

        base = s * rps_acc
        for k0 in range(0, rps_acc, C):
            sz = min(C, rps_acc - k0)
            pltpu.sync_copy(rows_v.at[pl.ds(0, sz)],
                            acc_sh.at[pl.ds(base + k0, sz)])
        plsc.subcore_barrier()

        @pl.loop(0, nch)
        def _acc(j):
            eb = (wid * nch + j) * C
            pltpu.sync_copy(srcs.at[pl.ds(eb, C)], src_v)
            pltpu.sync_copy(dsts.at[pl.ds(eb, C)], dst_v)
            pltpu.async_copy(table.at[src_v], rows_v, sem).wait()
            pltpu.sync_copy(rows_v, acc_sh.at[dst_v], add=True)

        plsc.subcore_barrier()

        ob = s * rps_out
        pltpu.sync_copy(acc_sh.at[pl.ds(ob, rps_out)],
                        acc_out.at[c, pl.ds(ob, rps_out)])
        if rem_out:
            @pl.when(s == 0)
            def _tail():
                rb = NS * rps_out
                pltpu.sync_copy(acc_sh.at[pl.ds(rb, rem_out)],
                                acc_out.at[c, pl.ds(rb, rem_out)])

    return pl.kernel(
        body,
        out_type=[jax.ShapeDtypeStruct((NC, n_nodes, d), jnp.float32)],
        mesh=mesh,
        scratch_types=[
            pltpu.VMEM((C,), jnp.int32),
            pltpu.VMEM((C,), jnp.int32),
            pltpu.VMEM((C, d), jnp.float32),
            pltpu.SemaphoreType.DMA,
            pltpu.VMEM_SHARED((na, d), jnp.float32),
        ],
        compiler_params=pltpu.CompilerParams(use_tc_tiling_on_sc=False))


def _tc_mid(x_ref, ws1_ref, wn1_ref, b1_ref, ws2_ref, wn2_ref, b2_ref,
            acc_ref, p2_ref, s2_ref, inv_ref):
    d_in = x_ref.shape[1]
    deg = jnp.maximum(acc_ref[0, :, d_in:d_in + 1] + acc_ref[1, :, d_in:d_in + 1],
                      1.0)
    inv = 1.0 / deg
    mean1 = (acc_ref[0, :, :d_in] + acc_ref[1, :, :d_in]) * inv
    h1 = _dot(x_ref[...], ws1_ref[...]) + _dot(mean1, wn1_ref[...]) + b1_ref[...]
    h1 = jnp.maximum(h1, 0.0)
    p2_ref[...] = _dot(h1, wn2_ref[...])
    s2_ref[...] = _dot(h1, ws2_ref[...]) + b2_ref[...]
    inv_ref[...] = jnp.broadcast_to(inv, inv_ref.shape)


def _tc_final(s2_ref, acc_ref, inv_ref, o_ref):
    o_ref[...] = s2_ref[...] + (acc_ref[0] + acc_ref[1]) * inv_ref[:, 0:1]


def kernel(inputs, edge_index, W_self1, W_neigh1, b1, W_self2, W_neigh2, b2):
    n, d_in = inputs.shape
    d_hid = W_self1.shape[1]
    d_out = W_self2.shape[1]
    e = edge_index.shape[1]
    assert e % NW == 0
    epw = e // NW
    nch = -(-epw // C)
    npad = nch * C - epw
    d_aug = d_in + 16

    src = edge_index[0].reshape(NW, epw)
    dst = edge_index[1].reshape(NW, epw)
    pad_src = jnp.broadcast_to(jnp.arange(npad, dtype=jnp.int32) % PAD_SPREAD,
                               (NW, npad))
    pad_dst = pad_src + n
    srcs = jnp.concatenate([src, pad_src], axis=1).reshape(NW * nch * C)
    dsts = jnp.concatenate([dst, pad_dst], axis=1).reshape(NW * nch * C)

    xaug = jnp.concatenate(
        [inputs, jnp.ones((n, d_aug - d_in), jnp.float32)], axis=1)
    (acc1,) = _make_sc_accum(n, d_aug, nch)(xaug, srcs, dsts)

    R = 400
    grid = (n // R,)
    p2, s2, inv = pl.pallas_call(
        _tc_mid,
        grid=grid,
        in_specs=[
            pl.BlockSpec((R, d_in), lambda i: (i, 0)),
            pl.BlockSpec((d_in, d_hid), lambda i: (0, 0)),
            pl.BlockSpec((d_in, d_hid), lambda i: (0, 0)),
            pl.BlockSpec((1, d_hid), lambda i: (0, 0)),
            pl.BlockSpec((d_hid, d_out), lambda i: (0, 0)),
            pl.BlockSpec((d_hid, d_out), lambda i: (0, 0)),
            pl.BlockSpec((1, d_out), lambda i: (0, 0)),
            pl.BlockSpec((NC, R, d_aug), lambda i: (0, i, 0)),
        ],
        out_specs=[
            pl.BlockSpec((R, d_out), lambda i: (i, 0)),
            pl.BlockSpec((R, d_out), lambda i: (i, 0)),
            pl.BlockSpec((R, 8), lambda i: (i, 0)),
        ],
        out_shape=[
            jax.ShapeDtypeStruct((n, d_out), jnp.float32),
            jax.ShapeDtypeStruct((n, d_out), jnp.float32),
            jax.ShapeDtypeStruct((n, 8), jnp.float32),
        ],
    )(inputs, W_self1, W_neigh1, b1.reshape(1, d_hid), W_self2, W_neigh2,
      b2.reshape(1, d_out), acc1)

    (acc2,) = _make_sc_accum(n, d_out, nch)(p2, srcs, dsts)

    out = pl.pallas_call(
        _tc_final,
        grid=grid,
        in_specs=[
            pl.BlockSpec((R, d_out), lambda i: (i, 0)),
            pl.BlockSpec((NC, R, d_out), lambda i: (0, i, 0)),
            pl.BlockSpec((R, 8), lambda i: (i, 0)),
        ],
        out_specs=pl.BlockSpec((R, d_out), lambda i: (i, 0)),
        out_shape=jax.ShapeDtypeStruct((n, d_out), jnp.float32),
    )(s2, acc2, inv)
    return out


# speedup vs baseline: 9.8367x; 1.5550x over previous
import jax
import jax.numpy as jnp
from jax import lax
from jax.experimental import pallas as pl
from jax.experimental.pallas import tpu as pltpu
from jax.experimental.pallas import tpu_sc as plsc

NC = 2
NS = 16
NW = NC * NS
C = 128
PAD_SPREAD = 64

_HIGH = jax.lax.Precision.HIGHEST


def _dot(a, b):
    return jax.lax.dot_general(a, b, (((1,), (0,)), ((), ())),
                               precision=_HIGH,
                               preferred_element_type=jnp.float32)


def _make_sc_accum(n_nodes, d, nch):
    na = n_nodes + PAD_SPREAD
    assert na % NS == 0 and d % 16 == 0 and (d * 4) % 64 == 0
    rps_acc = na // NS
    rps_out = (n_nodes // NS) & ~7
    rem_out = n_nodes - NS * rps_out
    mesh = plsc.VectorSubcoreMesh(core_axis_name="c", subcore_axis_name="s",
                                  num_cores=NC, num_subcores=NS)

    assert nch % 2 == 0 and nch >= 4

    def body(table, srcs, dsts, acc_out,
             src_v, dst_v, rows0, rows1, sem_g, sem_i, acc_sh):
        c = lax.axis_index("c")
        s = lax.axis_index("s")
        wid = s * NC + c
        z16 = jnp.zeros((16,), jnp.float32)

        @pl.loop(0, C)
        def _fill(r):
            for k in range(d // 16):
                rows0[r, pl.ds(k * 16, 16)] = z16

        base = s * rps_acc
        for k0 in range(0, rps_acc, C):
            sz = min(C, rps_acc - k0)
            pltpu.sync_copy(rows0.at[pl.ds(0, sz)],
                            acc_sh.at[pl.ds(base + k0, sz)])
        plsc.subcore_barrier()

        def idx_start(j, slot):
            eb = (wid * nch + j) * C
            pltpu.async_copy(srcs.at[pl.ds(eb, C)], src_v.at[slot], sem_i)
            pltpu.async_copy(dsts.at[pl.ds(eb, C)], dst_v.at[slot], sem_i)

        def idx_wait(slot):
            pltpu.make_async_copy(srcs.at[pl.ds(0, C)], src_v.at[slot],
                                  sem_i).wait()
            pltpu.make_async_copy(dsts.at[pl.ds(0, C)], dst_v.at[slot],
                                  sem_i).wait()

        def gather_start(slot, rows):
            pltpu.async_copy(table.at[src_v.at[slot]], rows, sem_g)

        def gather_wait(rows):
            pltpu.make_async_copy(table.at[pl.ds(0, C)], rows, sem_g).wait()

        def scatter(rows, slot):
            pltpu.sync_copy(rows, acc_sh.at[dst_v.at[slot]], add=True)

        idx_start(0, 0)
        idx_wait(0)
        gather_start(0, rows0)
        idx_start(1, 1)

        @pl.loop(0, nch // 2)
        def _pair(p):
            a = 2 * p
            gather_wait(rows0)
            idx_wait(1)
            gather_start(1, rows1)
            scatter(rows0, 0)
            @pl.when(a + 2 < nch)
            def _pf_a():
                idx_start(a + 2, 0)
            gather_wait(rows1)
            @pl.when(a + 2 < nch)
            def _g_a():
                idx_wait(0)
                gather_start(0, rows0)
            scatter(rows1, 1)
            @pl.when(a + 3 < nch)
            def _pf_b():
                idx_start(a + 3, 1)

        plsc.subcore_barrier()

        ob = s * rps_out
        pltpu.sync_copy(acc_sh.at[pl.ds(ob, rps_out)],
                        acc_out.at[c, pl.ds(ob, rps_out)])
        if rem_out:
            @pl.when(s == 0)
            def _tail():
                rb = NS * rps_out
                pltpu.sync_copy(acc_sh.at[pl.ds(rb, rem_out)],
                                acc_out.at[c, pl.ds(rb, rem_out)])

    return pl.kernel(
        body,
        out_type=[jax.ShapeDtypeStruct((NC, n_nodes, d), jnp.float32)],
        mesh=mesh,
        scratch_types=[
            pltpu.VMEM((2, C), jnp.int32),
            pltpu.VMEM((2, C), jnp.int32),
            pltpu.VMEM((C, d), jnp.float32),
            pltpu.VMEM((C, d), jnp.float32),
            pltpu.SemaphoreType.DMA,
            pltpu.SemaphoreType.DMA,
            pltpu.VMEM_SHARED((na, d), jnp.float32),
        ],
        compiler_params=pltpu.CompilerParams(use_tc_tiling_on_sc=False))


def _tc_mid(x_ref, ws1_ref, wn1_ref, b1_ref, ws2_ref, wn2_ref, b2_ref,
            acc_ref, p2_ref, s2_ref, inv_ref):
    d_in = x_ref.shape[1]
    deg = jnp.maximum(acc_ref[0, :, d_in:d_in + 1] + acc_ref[1, :, d_in:d_in + 1],
                      1.0)
    inv = 1.0 / deg
    mean1 = (acc_ref[0, :, :d_in] + acc_ref[1, :, :d_in]) * inv
    h1 = _dot(x_ref[...], ws1_ref[...]) + _dot(mean1, wn1_ref[...]) + b1_ref[...]
    h1 = jnp.maximum(h1, 0.0)
    p2_ref[...] = _dot(h1, wn2_ref[...])
    s2_ref[...] = _dot(h1, ws2_ref[...]) + b2_ref[...]
    inv_ref[...] = jnp.broadcast_to(inv, inv_ref.shape)


def _tc_final(s2_ref, acc_ref, inv_ref, o_ref):
    o_ref[...] = s2_ref[...] + (acc_ref[0] + acc_ref[1]) * inv_ref[:, 0:1]


def kernel(inputs, edge_index, W_self1, W_neigh1, b1, W_self2, W_neigh2, b2):
    n, d_in = inputs.shape
    d_hid = W_self1.shape[1]
    d_out = W_self2.shape[1]
    e = edge_index.shape[1]
    assert e % NW == 0
    epw = e // NW
    nch = -(-epw // C)
    nch += nch % 2
    npad = nch * C - epw
    d_aug = d_in + 16

    src = edge_index[0].reshape(NW, epw)
    dst = edge_index[1].reshape(NW, epw)
    pad_src = jnp.broadcast_to(jnp.arange(npad, dtype=jnp.int32) % PAD_SPREAD,
                               (NW, npad))
    pad_dst = pad_src + n
    srcs = jnp.concatenate([src, pad_src], axis=1).reshape(NW * nch * C)
    dsts = jnp.concatenate([dst, pad_dst], axis=1).reshape(NW * nch * C)

    xaug = jnp.concatenate(
        [inputs, jnp.ones((n, d_aug - d_in), jnp.float32)], axis=1)
    (acc1,) = _make_sc_accum(n, d_aug, nch)(xaug, srcs, dsts)

    R = 400
    grid = (n // R,)
    p2, s2, inv = pl.pallas_call(
        _tc_mid,
        grid=grid,
        in_specs=[
            pl.BlockSpec((R, d_in), lambda i: (i, 0)),
            pl.BlockSpec((d_in, d_hid), lambda i: (0, 0)),
            pl.BlockSpec((d_in, d_hid), lambda i: (0, 0)),
            pl.BlockSpec((1, d_hid), lambda i: (0, 0)),
            pl.BlockSpec((d_hid, d_out), lambda i: (0, 0)),
            pl.BlockSpec((d_hid, d_out), lambda i: (0, 0)),
            pl.BlockSpec((1, d_out), lambda i: (0, 0)),
            pl.BlockSpec((NC, R, d_aug), lambda i: (0, i, 0)),
        ],
        out_specs=[
            pl.BlockSpec((R, d_out), lambda i: (i, 0)),
            pl.BlockSpec((R, d_out), lambda i: (i, 0)),
            pl.BlockSpec((R, 8), lambda i: (i, 0)),
        ],
        out_shape=[
            jax.ShapeDtypeStruct((n, d_out), jnp.float32),
            jax.ShapeDtypeStruct((n, d_out), jnp.float32),
            jax.ShapeDtypeStruct((n, 8), jnp.float32),
        ],
    )(inputs, W_self1, W_neigh1, b1.reshape(1, d_hid), W_self2, W_neigh2,
      b2.reshape(1, d_out), acc1)

    (acc2,) = _make_sc_accum(n, d_out, nch)(p2, srcs, dsts)

    out = pl.pallas_call(
        _tc_final,
        grid=grid,
        in_specs=[
            pl.BlockSpec((R, d_out), lambda i: (i, 0)),
            pl.BlockSpec((NC, R, d_out), lambda i: (0, i, 0)),
            pl.BlockSpec((R, 8), lambda i: (i, 0)),
        ],
        out_specs=pl.BlockSpec((R, d_out), lambda i: (i, 0)),
        out_shape=jax.ShapeDtypeStruct((n, d_out), jnp.float32),
    )(s2, acc2, inv)
    return out
